# baseline (speedup 1.0000x reference)
import jax
import jax.numpy as jnp
from jax import lax
from jax.experimental import pallas as pl
from jax.experimental.pallas import tpu as pltpu
from jax.experimental.pallas import tpu_sc as plsc

_BATCH = 4096
_FIELDS = 26
_NC = 2
_NS = 16
_NW = _NC * _NS
_BPW = _BATCH // _NW
_IPW = _BPW * _FIELDS
_L = 16


_H = _IPW // 2
_HF = _FIELDS // 2


def _sc_body(xt_hbm, w_hbm, out_hbm, idx_v, vals_v, out_v, sem, sem2):
    wid = lax.axis_index("s") * _NC + lax.axis_index("c")
    w1 = w_hbm.at[0]
    base = wid * _BPW
    pltpu.sync_copy(xt_hbm.at[:, pl.ds(base, _BPW)], idx_v)
    gs = [
        pltpu.async_copy(
            w1.at[idx_v.at[j]], vals_v.at[pl.ds(j * _BPW, _BPW)],
            sem if j < _HF else sem2)
        for j in range(_FIELDS)
    ]
    for g in gs[:_HF]:
        pass
    g_a = gs[0]
    g_b = gs[-1]
    for g in gs[:_HF]:
        g.wait()
    accs = []
    for chunk in range(_BPW // _L):
        acc = vals_v[pl.ds(chunk * _L, _L)]
        for j in range(1, _HF):
            acc = acc + vals_v[pl.ds(j * _BPW + chunk * _L, _L)]
        accs.append(acc)
    for g in gs[_HF:]:
        g.wait()
    for chunk in range(_BPW // _L):
        acc = accs[chunk]
        for j in range(_HF, _FIELDS):
            acc = acc + vals_v[pl.ds(j * _BPW + chunk * _L, _L)]
        out_v[pl.ds(chunk * _L, _L)] = acc
    pltpu.sync_copy(out_v, out_hbm.at[pl.ds(base, _BPW)])


def _flatten_table(fc_weight):
    return fc_weight.reshape(1, -1)


def kernel(x, fc_weight, bias):
    xt = x.astype(jnp.int32).T
    mesh = plsc.VectorSubcoreMesh(core_axis_name="c", subcore_axis_name="s")
    out = pl.kernel(
        _sc_body,
        out_type=jax.ShapeDtypeStruct((_BATCH,), jnp.float32),
        mesh=mesh,
        scratch_types=[
            pltpu.VMEM((_FIELDS, _BPW), jnp.int32),
            pltpu.VMEM((_IPW,), jnp.float32),
            pltpu.VMEM((_BPW,), jnp.float32),
            pltpu.SemaphoreType.DMA,
            pltpu.SemaphoreType.DMA,
        ],
    )(xt, _flatten_table(fc_weight))
    return out.reshape(_BATCH, 1) + bias


# speedup vs baseline: 1.0291x; 1.0291x over previous
import jax
import jax.numpy as jnp
from jax import lax
from jax.experimental import pallas as pl
from jax.experimental.pallas import tpu as pltpu
from jax.experimental.pallas import tpu_sc as plsc

_BATCH = 4096
_FIELDS = 26
_NC = 2
_NS = 16
_NW = _NC * _NS
_BPW = _BATCH // _NW
_IPW = _BPW * _FIELDS
_L = 16


_H = _IPW // 2
_HF = _FIELDS // 2


def _sc_body(xt_hbm, w_hbm, b_hbm, out_hbm, idx_v, vals_v, out_v, brow_v, sem, sem2, sem3):
    wid = lax.axis_index("s") * _NC + lax.axis_index("c")
    w1 = w_hbm.at[0]
    bg = pltpu.async_copy(b_hbm.at[0], brow_v, sem3)
    base = wid * _IPW
    cp_a = pltpu.async_copy(
        xt_hbm.at[pl.ds(base, _H)], idx_v.at[pl.ds(0, _H)], sem)
    cp_b = pltpu.async_copy(
        xt_hbm.at[pl.ds(base + _H, _H)], idx_v.at[pl.ds(_H, _H)], sem2)
    cp_a.wait()
    g_a = pltpu.async_copy(
        w1.at[idx_v.at[pl.ds(0, _H)]], vals_v.at[pl.ds(0, _H)], sem)
    cp_b.wait()
    g_b = pltpu.async_copy(
        w1.at[idx_v.at[pl.ds(_H, _H)]], vals_v.at[pl.ds(_H, _H)], sem2)
    bg.wait()
    bias_vec = brow_v[pl.ds(0, _L)]
    g_a.wait()
    accs = []
    for chunk in range(_BPW // _L):
        acc = bias_vec
        for j in range(0, _HF):
            acc = acc + vals_v[pl.ds(j * _BPW + chunk * _L, _L)]
        accs.append(acc)
    g_b.wait()
    for chunk in range(_BPW // _L):
        acc = accs[chunk]
        for j in range(_HF, _FIELDS):
            acc = acc + vals_v[pl.ds(j * _BPW + chunk * _L, _L)]
        out_v[pl.ds(chunk * _L, _L)] = acc
    pltpu.sync_copy(out_v, out_hbm.at[pl.ds(wid * _BPW, _BPW)])


def _flatten_table(fc_weight):
    return fc_weight.reshape(1, -1)


def kernel(x, fc_weight, bias):
    xt = jnp.transpose(
        x.astype(jnp.int32).reshape(_NW, _BPW, _FIELDS), (0, 2, 1)
    ).reshape(-1)
    mesh = plsc.VectorSubcoreMesh(core_axis_name="c", subcore_axis_name="s")
    out = pl.kernel(
        _sc_body,
        out_type=jax.ShapeDtypeStruct((_BATCH,), jnp.float32),
        mesh=mesh,
        scratch_types=[
            pltpu.VMEM((_IPW,), jnp.int32),
            pltpu.VMEM((_IPW,), jnp.float32),
            pltpu.VMEM((_BPW,), jnp.float32),
            pltpu.VMEM((_L,), jnp.float32),
            pltpu.SemaphoreType.DMA,
            pltpu.SemaphoreType.DMA,
            pltpu.SemaphoreType.DMA,
        ],
    )(xt, _flatten_table(fc_weight), jnp.broadcast_to(bias.reshape(1, 1), (1, _L)))
    return out.reshape(_BATCH, 1)
